# Initial kernel scaffold; baseline (speedup 1.0000x reference)
import functools

import jax
import jax.numpy as jnp
from jax import lax
from jax.experimental import pallas as pl
from jax.experimental.pallas import tpu as pltpu
from jax.experimental.pallas import tpu_sc as plsc

N = 10000
E = 320000
D = 128
R = 8

NC = 2
NS = 16
LANE = 128
CH = 80
EPAD = NC * NS * CH * LANE
NPAD = 10016
BN = 1000


def _pre_body(x_ref, w_ref, lw_ref, hw_ref, loop_ref):
    x = x_ref[...]
    for r in range(R):
        hw_ref[:, r, :] = jnp.dot(x, w_ref[r], preferred_element_type=jnp.float32)
    loop_ref[...] = jnp.dot(x, lw_ref[...], preferred_element_type=jnp.float32)


_pre = pl.pallas_call(
    _pre_body,
    grid=(N // BN,),
    in_specs=[
        pl.BlockSpec((BN, D), lambda i: (i, 0)),
        pl.BlockSpec((R, D, D), lambda i: (0, 0, 0)),
        pl.BlockSpec((D, D), lambda i: (0, 0)),
    ],
    out_specs=[
        pl.BlockSpec((BN, R, D), lambda i: (i, 0, 0)),
        pl.BlockSpec((BN, D), lambda i: (i, 0)),
    ],
    out_shape=[
        jax.ShapeDtypeStruct((N, R, D), jnp.float32),
        jax.ShapeDtypeStruct((N, D), jnp.float32),
    ],
)


def _post_body(parts_ref, loop_ref, g_ref, b_ref, bias_ref, o_ref, *, act):
    aggv = parts_ref[0] + parts_ref[1]
    mean = jnp.mean(aggv, axis=-1, keepdims=True)
    xc = aggv - mean
    var = jnp.mean(xc * xc, axis=-1, keepdims=True)
    h = (xc * lax.rsqrt(var + 1e-5) * g_ref[...] + b_ref[...]
         + bias_ref[...] + loop_ref[...])
    o_ref[...] = jnp.maximum(h, 0.0) if act else h


def _make_post(act):
    return pl.pallas_call(
        functools.partial(_post_body, act=act),
        grid=(N // BN,),
        in_specs=[
            pl.BlockSpec((2, BN, D), lambda i: (0, i, 0)),
            pl.BlockSpec((BN, D), lambda i: (i, 0)),
            pl.BlockSpec((1, D), lambda i: (0, 0)),
            pl.BlockSpec((1, D), lambda i: (0, 0)),
            pl.BlockSpec((1, D), lambda i: (0, 0)),
        ],
        out_specs=pl.BlockSpec((BN, D), lambda i: (i, 0)),
        out_shape=jax.ShapeDtypeStruct((N, D), jnp.float32),
    )


_post_act = _make_post(True)
_post_noact = _make_post(False)


def _sc_body(hw_hbm, idx_hbm, dst_hbm, zeros_hbm, out_hbm,
             idx_v, dst_v, buf, agg, sem):
    c = lax.axis_index("c")
    s = lax.axis_index("s")
    zrows = NPAD // NS
    pltpu.sync_copy(zeros_hbm, agg.at[pl.ds(s * zrows, zrows)])
    pltpu.sync_copy(idx_hbm.at[c, s], idx_v)
    pltpu.sync_copy(dst_hbm.at[c, s], dst_v)
    plsc.subcore_barrier()

    def body(j, carry):
        pltpu.async_copy(hw_hbm.at[idx_v.at[j]], buf, sem).wait()
        pltpu.sync_copy(buf, agg.at[dst_v.at[j]], add=True)
        return carry

    lax.fori_loop(0, CH, body, 0)
    plsc.subcore_barrier()
    orows = N // NS
    pltpu.sync_copy(agg.at[pl.ds(s * orows, orows)],
                    out_hbm.at[c, pl.ds(s * orows, orows)])


_sc_agg = pl.kernel(
    _sc_body,
    out_type=jax.ShapeDtypeStruct((NC, N, D), jnp.float32),
    mesh=plsc.VectorSubcoreMesh(core_axis_name="c", subcore_axis_name="s"),
    scratch_types=[
        pltpu.VMEM((CH, LANE), jnp.int32),
        pltpu.VMEM((CH, LANE), jnp.int32),
        pltpu.VMEM((LANE, D), jnp.float32),
        pltpu.VMEM_SHARED((NPAD, D), jnp.float32),
        pltpu.SemaphoreType.DMA,
    ],
)


def kernel(feat, edge_index, etypes,
           W0, bias0, loopW0, ln_g0, ln_b0,
           W1, bias1, loopW1, ln_g1, ln_b1,
           W2, bias2, loopW2, ln_g2, ln_b2):
    src = edge_index[0]
    dst = edge_index[1]
    flat = src * R + etypes
    pad = EPAD - E
    flat_p = jnp.concatenate([flat, jnp.zeros((pad,), jnp.int32)])
    dst_p = jnp.concatenate([dst, jnp.full((pad,), N, jnp.int32)])
    idx3 = flat_p.reshape(NC, NS, CH, LANE)
    dst3 = dst_p.reshape(NC, NS, CH, LANE)
    zeros = jnp.zeros((NPAD // NS, D), jnp.float32)

    h = feat
    layers = (
        (W0, bias0, loopW0, ln_g0, ln_b0, True),
        (W1, bias1, loopW1, ln_g1, ln_b1, True),
        (W2, bias2, loopW2, ln_g2, ln_b2, False),
    )
    for W, bias, loopW, g, b, act in layers:
        hw, loop = _pre(h, W, loopW)
        parts = _sc_agg(hw.reshape(N * R, D), idx3, dst3, zeros)
        post = _post_act if act else _post_noact
        h = post(parts, loop, g.reshape(1, D), b.reshape(1, D),
                 bias.reshape(1, D))
    return h


# speedup vs baseline: 11.4630x; 11.4630x over previous
import functools

import jax
import jax.numpy as jnp
from jax import lax
from jax.experimental import pallas as pl
from jax.experimental.pallas import tpu as pltpu
from jax.experimental.pallas import tpu_sc as plsc

N = 10000
E = 320000
D = 128
R = 8

NC = 2
NS = 16
LANE = 128
CH = 80
EPAD = NC * NS * CH * LANE
NPAD = 10112
BN = 1000


def _pre_body(x_ref, w_ref, lw_ref, hw_ref, loop_ref):
    x = x_ref[...]
    for r in range(R):
        hw_ref[:, r, :] = jnp.dot(x, w_ref[r], preferred_element_type=jnp.float32)
    loop_ref[...] = jnp.dot(x, lw_ref[...], preferred_element_type=jnp.float32)


_pre = pl.pallas_call(
    _pre_body,
    grid=(N // BN,),
    in_specs=[
        pl.BlockSpec((BN, D), lambda i: (i, 0)),
        pl.BlockSpec((R, D, D), lambda i: (0, 0, 0)),
        pl.BlockSpec((D, D), lambda i: (0, 0)),
    ],
    out_specs=[
        pl.BlockSpec((BN, R, D), lambda i: (i, 0, 0)),
        pl.BlockSpec((BN, D), lambda i: (i, 0)),
    ],
    out_shape=[
        jax.ShapeDtypeStruct((N, R, D), jnp.float32),
        jax.ShapeDtypeStruct((N, D), jnp.float32),
    ],
)


def _post_body(parts_ref, loop_ref, g_ref, b_ref, bias_ref, o_ref, *, act):
    aggv = parts_ref[0] + parts_ref[1]
    mean = jnp.mean(aggv, axis=-1, keepdims=True)
    xc = aggv - mean
    var = jnp.mean(xc * xc, axis=-1, keepdims=True)
    h = (xc * lax.rsqrt(var + 1e-5) * g_ref[...] + b_ref[...]
         + bias_ref[...] + loop_ref[...])
    o_ref[...] = jnp.maximum(h, 0.0) if act else h


def _make_post(act):
    return pl.pallas_call(
        functools.partial(_post_body, act=act),
        grid=(N // BN,),
        in_specs=[
            pl.BlockSpec((2, BN, D), lambda i: (0, i, 0)),
            pl.BlockSpec((BN, D), lambda i: (i, 0)),
            pl.BlockSpec((1, D), lambda i: (0, 0)),
            pl.BlockSpec((1, D), lambda i: (0, 0)),
            pl.BlockSpec((1, D), lambda i: (0, 0)),
        ],
        out_specs=pl.BlockSpec((BN, D), lambda i: (i, 0)),
        out_shape=jax.ShapeDtypeStruct((N, D), jnp.float32),
    )


_post_act = _make_post(True)
_post_noact = _make_post(False)


def _sc_body(hw_hbm, idx_hbm, dst_hbm, zeros_hbm, out_hbm,
             idx_v, dst_v, buf, agg, sem):
    c = lax.axis_index("c")
    s = lax.axis_index("s")
    zrows = NPAD // NS
    pltpu.sync_copy(zeros_hbm, agg.at[pl.ds(s * zrows, zrows)])
    pltpu.sync_copy(idx_hbm.at[c, s], idx_v)
    pltpu.sync_copy(dst_hbm.at[c, s], dst_v)
    plsc.subcore_barrier()

    def body(j, carry):
        pltpu.async_copy(hw_hbm.at[idx_v.at[j]], buf, sem).wait()
        pltpu.sync_copy(buf, agg.at[dst_v.at[j]], add=True)
        return carry

    lax.fori_loop(0, CH, body, 0)
    plsc.subcore_barrier()
    orows = NPAD // NS
    pltpu.sync_copy(agg.at[pl.ds(s * orows, orows)],
                    out_hbm.at[c, pl.ds(s * orows, orows)])


@functools.cache
def _get_sc_agg():
    return pl.kernel(
        _sc_body,
        out_type=jax.ShapeDtypeStruct((NC, NPAD, D), jnp.float32),
        mesh=plsc.VectorSubcoreMesh(core_axis_name="c", subcore_axis_name="s",
                                    num_cores=NC, num_subcores=NS),
        scratch_types=[
            pltpu.VMEM((CH, LANE), jnp.int32),
            pltpu.VMEM((CH, LANE), jnp.int32),
            pltpu.VMEM((LANE, D), jnp.float32),
            pltpu.VMEM_SHARED((NPAD, D), jnp.float32),
            pltpu.SemaphoreType.DMA,
        ],
    )


def kernel(feat, edge_index, etypes,
           W0, bias0, loopW0, ln_g0, ln_b0,
           W1, bias1, loopW1, ln_g1, ln_b1,
           W2, bias2, loopW2, ln_g2, ln_b2):
    src = edge_index[0]
    dst = edge_index[1]
    flat = src * R + etypes
    pad = EPAD - E
    flat_p = jnp.concatenate([flat, jnp.zeros((pad,), jnp.int32)])
    dst_p = jnp.concatenate([dst, jnp.full((pad,), N, jnp.int32)])
    idx3 = flat_p.reshape(NC, NS, CH, LANE)
    dst3 = dst_p.reshape(NC, NS, CH, LANE)
    zeros = jnp.zeros((NPAD // NS, D), jnp.float32)

    h = feat
    layers = (
        (W0, bias0, loopW0, ln_g0, ln_b0, True),
        (W1, bias1, loopW1, ln_g1, ln_b1, True),
        (W2, bias2, loopW2, ln_g2, ln_b2, False),
    )
    for W, bias, loopW, g, b, act in layers:
        hw, loop = _pre(h, W, loopW)
        parts = _get_sc_agg()(hw.reshape(N * R, D), idx3, dst3, zeros)
        post = _post_act if act else _post_noact
        h = post(parts, loop, g.reshape(1, D), b.reshape(1, D),
                 bias.reshape(1, D))
    return h


# speedup vs baseline: 12.4394x; 1.0852x over previous
import functools

import jax
import jax.numpy as jnp
from jax import lax
from jax.experimental import pallas as pl
from jax.experimental.pallas import tpu as pltpu
from jax.experimental.pallas import tpu_sc as plsc

N = 10000
E = 320000
D = 128
R = 8

NC = 2
NS = 16
LANE = 128
CH = 80
EPAD = NC * NS * CH * LANE
NPAD = 10112
BN = 1000


def _pre_body(x_ref, w_ref, lw_ref, hw_ref, loop_ref):
    x = x_ref[...]
    for r in range(R):
        hw_ref[:, r, :] = jnp.dot(x, w_ref[r], preferred_element_type=jnp.float32)
    loop_ref[...] = jnp.dot(x, lw_ref[...], preferred_element_type=jnp.float32)


_pre = pl.pallas_call(
    _pre_body,
    grid=(N // BN,),
    in_specs=[
        pl.BlockSpec((BN, D), lambda i: (i, 0)),
        pl.BlockSpec((R, D, D), lambda i: (0, 0, 0)),
        pl.BlockSpec((D, D), lambda i: (0, 0)),
    ],
    out_specs=[
        pl.BlockSpec((BN, R, D), lambda i: (i, 0, 0)),
        pl.BlockSpec((BN, D), lambda i: (i, 0)),
    ],
    out_shape=[
        jax.ShapeDtypeStruct((N, R, D), jnp.float32),
        jax.ShapeDtypeStruct((N, D), jnp.float32),
    ],
)


def _post_body(parts_ref, loop_ref, g_ref, b_ref, bias_ref, o_ref, *, act):
    aggv = parts_ref[0] + parts_ref[1]
    mean = jnp.mean(aggv, axis=-1, keepdims=True)
    xc = aggv - mean
    var = jnp.mean(xc * xc, axis=-1, keepdims=True)
    h = (xc * lax.rsqrt(var + 1e-5) * g_ref[...] + b_ref[...]
         + bias_ref[...] + loop_ref[...])
    o_ref[...] = jnp.maximum(h, 0.0) if act else h


def _make_post(act):
    return pl.pallas_call(
        functools.partial(_post_body, act=act),
        grid=(N // BN,),
        in_specs=[
            pl.BlockSpec((2, BN, D), lambda i: (0, i, 0)),
            pl.BlockSpec((BN, D), lambda i: (i, 0)),
            pl.BlockSpec((1, D), lambda i: (0, 0)),
            pl.BlockSpec((1, D), lambda i: (0, 0)),
            pl.BlockSpec((1, D), lambda i: (0, 0)),
        ],
        out_specs=pl.BlockSpec((BN, D), lambda i: (i, 0)),
        out_shape=jax.ShapeDtypeStruct((N, D), jnp.float32),
    )


_post_act = _make_post(True)
_post_noact = _make_post(False)


NBUF = 2
RING = 4


def _sc_body(hw_hbm, idx_hbm, dst_hbm, zeros_hbm, out_hbm,
             dst_v, *rest):
    islots = rest[0:RING]
    isems = rest[RING:2 * RING]
    bufs = rest[2 * RING:2 * RING + NBUF]
    gsems = rest[2 * RING + NBUF:2 * RING + 2 * NBUF]
    ssems = rest[2 * RING + 2 * NBUF:2 * RING + 3 * NBUF]
    agg = rest[2 * RING + 3 * NBUF]
    c = lax.axis_index("c")
    s = lax.axis_index("s")
    zrows = NPAD // NS
    pltpu.sync_copy(zeros_hbm, agg.at[pl.ds(s * zrows, zrows)])
    pltpu.sync_copy(dst_hbm.at[c, s], dst_v)

    def load_idx(j, i):
        pltpu.async_copy(idx_hbm.at[c, s, j], islots[i], isems[i])

    def wait_i(i):
        pltpu.make_async_copy(idx_hbm.at[c, s, 0], islots[i], isems[i]).wait()

    def gather(i, b):
        pltpu.async_copy(hw_hbm.at[islots[i]], bufs[b], gsems[b])

    def wait_g(b):
        pltpu.make_async_copy(hw_hbm.at[islots[0]], bufs[b], gsems[b]).wait()

    def scatter(j, b):
        pltpu.async_copy(bufs[b], agg.at[dst_v.at[j]], ssems[b], add=True)

    def wait_s(b):
        pltpu.make_async_copy(bufs[b], agg.at[dst_v.at[0]], ssems[b]).wait()

    plsc.subcore_barrier()

    for i in range(RING):
        load_idx(i, i)
    wait_i(0)
    gather(0, 0)

    def body(k, carry):
        for u in range(RING):
            j = k * RING + u

            wait_g(u % NBUF)

            @pl.when(j + RING < CH)
            def _():
                load_idx(j + RING, u)

            scatter(j, u % NBUF)

            @pl.when(j >= 1)
            def _():
                wait_s((u + 1) % NBUF)

            @pl.when(j + 1 < CH)
            def _():
                wait_i((u + 1) % RING)
                gather((u + 1) % RING, (u + 1) % NBUF)

        return carry

    lax.fori_loop(0, CH // RING, body, 0)
    wait_s((CH - 1) % NBUF)
    plsc.subcore_barrier()
    orows = NPAD // NS
    pltpu.sync_copy(agg.at[pl.ds(s * orows, orows)],
                    out_hbm.at[c, pl.ds(s * orows, orows)])


@functools.cache
def _get_sc_agg():
    return pl.kernel(
        _sc_body,
        out_type=jax.ShapeDtypeStruct((NC, NPAD, D), jnp.float32),
        mesh=plsc.VectorSubcoreMesh(core_axis_name="c", subcore_axis_name="s",
                                    num_cores=NC, num_subcores=NS),
        scratch_types=(
            [pltpu.VMEM((CH, LANE), jnp.int32)]
            + [pltpu.VMEM((LANE,), jnp.int32) for _ in range(RING)]
            + [pltpu.SemaphoreType.DMA for _ in range(RING)]
            + [pltpu.VMEM((LANE, D), jnp.float32) for _ in range(NBUF)]
            + [pltpu.SemaphoreType.DMA for _ in range(2 * NBUF)]
            + [pltpu.VMEM_SHARED((NPAD, D), jnp.float32)]
        ),
    )


def kernel(feat, edge_index, etypes,
           W0, bias0, loopW0, ln_g0, ln_b0,
           W1, bias1, loopW1, ln_g1, ln_b1,
           W2, bias2, loopW2, ln_g2, ln_b2):
    src = edge_index[0]
    dst = edge_index[1]
    flat = src * R + etypes
    pad = EPAD - E
    flat_p = jnp.concatenate([flat, jnp.zeros((pad,), jnp.int32)])
    dst_p = jnp.concatenate([dst, jnp.full((pad,), N, jnp.int32)])
    idx3 = flat_p.reshape(NC, NS, CH, LANE)
    dst3 = dst_p.reshape(NC, NS, CH, LANE)
    zeros = jnp.zeros((NPAD // NS, D), jnp.float32)

    h = feat
    layers = (
        (W0, bias0, loopW0, ln_g0, ln_b0, True),
        (W1, bias1, loopW1, ln_g1, ln_b1, True),
        (W2, bias2, loopW2, ln_g2, ln_b2, False),
    )
    for W, bias, loopW, g, b, act in layers:
        hw, loop = _pre(h, W, loopW)
        parts = _get_sc_agg()(hw.reshape(N * R, D), idx3, dst3, zeros)
        post = _post_act if act else _post_noact
        h = post(parts, loop, g.reshape(1, D), b.reshape(1, D),
                 bias.reshape(1, D))
    return h


# speedup vs baseline: 13.1604x; 1.0580x over previous
import functools

import jax
import jax.numpy as jnp
from jax import lax
from jax.experimental import pallas as pl
from jax.experimental.pallas import tpu as pltpu
from jax.experimental.pallas import tpu_sc as plsc

N = 10000
E = 320000
D = 128
R = 8

NC = 2
NS = 16
LANE = 128
CH = 80
EPAD = NC * NS * CH * LANE
NPAD = 10112
BN = 1000


def _pre_body(x_ref, w_ref, lw_ref, hw_ref, loop_ref):
    x = x_ref[...]
    for r in range(R):
        hw_ref[:, r, :] = jnp.dot(x, w_ref[r], preferred_element_type=jnp.float32)
    loop_ref[...] = jnp.dot(x, lw_ref[...], preferred_element_type=jnp.float32)


_pre = pl.pallas_call(
    _pre_body,
    grid=(N // BN,),
    in_specs=[
        pl.BlockSpec((BN, D), lambda i: (i, 0)),
        pl.BlockSpec((R, D, D), lambda i: (0, 0, 0)),
        pl.BlockSpec((D, D), lambda i: (0, 0)),
    ],
    out_specs=[
        pl.BlockSpec((BN, R, D), lambda i: (i, 0, 0)),
        pl.BlockSpec((BN, D), lambda i: (i, 0)),
    ],
    out_shape=[
        jax.ShapeDtypeStruct((N, R, D), jnp.float32),
        jax.ShapeDtypeStruct((N, D), jnp.float32),
    ],
)


def _post_body(parts_ref, loop_ref, g_ref, b_ref, bias_ref, o_ref, *, act):
    aggv = parts_ref[0] + parts_ref[1]
    mean = jnp.mean(aggv, axis=-1, keepdims=True)
    xc = aggv - mean
    var = jnp.mean(xc * xc, axis=-1, keepdims=True)
    h = (xc * lax.rsqrt(var + 1e-5) * g_ref[...] + b_ref[...]
         + bias_ref[...] + loop_ref[...])
    o_ref[...] = jnp.maximum(h, 0.0) if act else h


def _make_post(act):
    return pl.pallas_call(
        functools.partial(_post_body, act=act),
        grid=(N // BN,),
        in_specs=[
            pl.BlockSpec((2, BN, D), lambda i: (0, i, 0)),
            pl.BlockSpec((BN, D), lambda i: (i, 0)),
            pl.BlockSpec((1, D), lambda i: (0, 0)),
            pl.BlockSpec((1, D), lambda i: (0, 0)),
            pl.BlockSpec((1, D), lambda i: (0, 0)),
        ],
        out_specs=pl.BlockSpec((BN, D), lambda i: (i, 0)),
        out_shape=jax.ShapeDtypeStruct((N, D), jnp.float32),
    )


_post_act = _make_post(True)
_post_noact = _make_post(False)


NBUF = 2
RING = 4


def _sc_body(hw_hbm, idx_hbm, dst_hbm, zeros_hbm, out_hbm,
             dst_v, *rest):
    islots = rest[0:RING]
    isems = rest[RING:2 * RING]
    bufs = rest[2 * RING:2 * RING + NBUF]
    gsems = rest[2 * RING + NBUF:2 * RING + 2 * NBUF]
    ssems = rest[2 * RING + 2 * NBUF:2 * RING + 3 * NBUF]
    agg = rest[2 * RING + 3 * NBUF]
    c = lax.axis_index("c")
    s = lax.axis_index("s")
    zrows = NPAD // NS
    pltpu.sync_copy(zeros_hbm, agg.at[pl.ds(s * zrows, zrows)])
    pltpu.sync_copy(dst_hbm.at[c, s], dst_v)

    def load_idx(j, i):
        pltpu.async_copy(idx_hbm.at[c, s, j], islots[i], isems[i])

    def wait_i(i):
        pltpu.make_async_copy(idx_hbm.at[c, s, 0], islots[i], isems[i]).wait()

    def gather(i, b):
        pltpu.async_copy(hw_hbm.at[islots[i]], bufs[b], gsems[b])

    def wait_g(b):
        pltpu.make_async_copy(hw_hbm.at[islots[0]], bufs[b], gsems[b]).wait()

    def scatter(j, b):
        pltpu.async_copy(bufs[b], agg.at[dst_v.at[j]], ssems[b], add=True)

    def wait_s(b):
        pltpu.make_async_copy(bufs[b], agg.at[dst_v.at[0]], ssems[b]).wait()

    plsc.subcore_barrier()

    for i in range(RING):
        load_idx(i, i)
    wait_i(0)
    gather(0, 0)

    def body(k, carry):
        for u in range(RING):
            j = k * RING + u

            wait_g(u % NBUF)

            @pl.when(j + RING < CH)
            def _():
                load_idx(j + RING, u)

            scatter(j, u % NBUF)

            @pl.when(j >= 1)
            def _():
                wait_s((u + 1) % NBUF)

            @pl.when(j + 1 < CH)
            def _():
                wait_i((u + 1) % RING)
                gather((u + 1) % RING, (u + 1) % NBUF)

        return carry

    lax.fori_loop(0, CH // RING, body, 0)
    wait_s((CH - 1) % NBUF)
    plsc.subcore_barrier()
    orows = NPAD // NS
    pltpu.sync_copy(agg.at[pl.ds(s * orows, orows)],
                    out_hbm.at[c, pl.ds(s * orows, orows)])


@functools.cache
def _get_sc_agg():
    return pl.kernel(
        _sc_body,
        out_type=jax.ShapeDtypeStruct((NC, NPAD, D), jnp.float32),
        mesh=plsc.VectorSubcoreMesh(core_axis_name="c", subcore_axis_name="s",
                                    num_cores=NC, num_subcores=NS),
        scratch_types=(
            [pltpu.VMEM((CH, LANE), jnp.int32)]
            + [pltpu.VMEM((LANE,), jnp.int32) for _ in range(RING)]
            + [pltpu.SemaphoreType.DMA for _ in range(RING)]
            + [pltpu.VMEM((LANE, D), jnp.float32) for _ in range(NBUF)]
            + [pltpu.SemaphoreType.DMA for _ in range(2 * NBUF)]
            + [pltpu.VMEM_SHARED((NPAD, D), jnp.float32)]
        ),
    )


def kernel(feat, edge_index, etypes,
           W0, bias0, loopW0, ln_g0, ln_b0,
           W1, bias1, loopW1, ln_g1, ln_b1,
           W2, bias2, loopW2, ln_g2, ln_b2):
    src = edge_index[0]
    dst = edge_index[1]
    flat = src * R + etypes
    nw = NC * NS
    per_w = E // nw
    pad_w = CH * LANE - per_w
    idx_pad = jnp.zeros((nw, pad_w), jnp.int32)
    dst_pad = jnp.broadcast_to(N + (jnp.arange(pad_w, dtype=jnp.int32)
                                    % (NPAD - N)), (nw, pad_w))
    flat_p = jnp.concatenate([flat.reshape(nw, per_w), idx_pad], axis=1)
    dst_p = jnp.concatenate([dst.reshape(nw, per_w), dst_pad], axis=1)
    idx3 = flat_p.reshape(NC, NS, CH, LANE)
    dst3 = dst_p.reshape(NC, NS, CH, LANE)
    zeros = jnp.zeros((NPAD // NS, D), jnp.float32)

    h = feat
    layers = (
        (W0, bias0, loopW0, ln_g0, ln_b0, True),
        (W1, bias1, loopW1, ln_g1, ln_b1, True),
        (W2, bias2, loopW2, ln_g2, ln_b2, False),
    )
    for W, bias, loopW, g, b, act in layers:
        hw, loop = _pre(h, W, loopW)
        parts = _get_sc_agg()(hw.reshape(N * R, D), idx3, dst3, zeros)
        post = _post_act if act else _post_noact
        h = post(parts, loop, g.reshape(1, D), b.reshape(1, D),
                 bias.reshape(1, D))
    return h


# speedup vs baseline: 32.5876x; 2.4762x over previous
import functools

import jax
import jax.numpy as jnp
from jax import lax
from jax.experimental import pallas as pl
from jax.experimental.pallas import tpu as pltpu
from jax.experimental.pallas import tpu_sc as plsc

N = 10000
E = 320000
D = 128
R = 8

NC = 2
NS = 16
LANE = 128
CH = 80
EPAD = NC * NS * CH * LANE
NPAD = 10112
BN = 1000


def _pre_body(x_ref, w_ref, lw_ref, hw_ref, loop_ref):
    x = x_ref[...]
    for r in range(R):
        hw_ref[:, r, :] = jnp.dot(x, w_ref[r], preferred_element_type=jnp.float32)
    loop_ref[...] = jnp.dot(x, lw_ref[...], preferred_element_type=jnp.float32)


_pre = pl.pallas_call(
    _pre_body,
    grid=(N // BN,),
    in_specs=[
        pl.BlockSpec((BN, D), lambda i: (i, 0)),
        pl.BlockSpec((R, D, D), lambda i: (0, 0, 0)),
        pl.BlockSpec((D, D), lambda i: (0, 0)),
    ],
    out_specs=[
        pl.BlockSpec((BN, R, D), lambda i: (i, 0, 0)),
        pl.BlockSpec((BN, D), lambda i: (i, 0)),
    ],
    out_shape=[
        jax.ShapeDtypeStruct((N, R, D), jnp.float32),
        jax.ShapeDtypeStruct((N, D), jnp.float32),
    ],
)


def _post_body(parts_ref, loop_ref, g_ref, b_ref, bias_ref, o_ref, *, act):
    aggv = parts_ref[0] + parts_ref[1]
    mean = jnp.mean(aggv, axis=-1, keepdims=True)
    xc = aggv - mean
    var = jnp.mean(xc * xc, axis=-1, keepdims=True)
    h = (xc * lax.rsqrt(var + 1e-5) * g_ref[...] + b_ref[...]
         + bias_ref[...] + loop_ref[...])
    o_ref[...] = jnp.maximum(h, 0.0) if act else h


def _make_post(act):
    return pl.pallas_call(
        functools.partial(_post_body, act=act),
        grid=(N // BN,),
        in_specs=[
            pl.BlockSpec((2, BN, D), lambda i: (0, i, 0)),
            pl.BlockSpec((BN, D), lambda i: (i, 0)),
            pl.BlockSpec((1, D), lambda i: (0, 0)),
            pl.BlockSpec((1, D), lambda i: (0, 0)),
            pl.BlockSpec((1, D), lambda i: (0, 0)),
        ],
        out_specs=pl.BlockSpec((BN, D), lambda i: (i, 0)),
        out_shape=jax.ShapeDtypeStruct((N, D), jnp.float32),
    )


_post_act = _make_post(True)
_post_noact = _make_post(False)


NBUF = 2
RING = 4


def _sc_body(hw_hbm, idx_hbm, dst_hbm, zeros_hbm, out_hbm,
             dst_v, *rest):
    islots = rest[0:RING]
    isems = rest[RING:2 * RING]
    bufs = rest[2 * RING:2 * RING + NBUF]
    gsems = rest[2 * RING + NBUF:2 * RING + 2 * NBUF]
    ssems = rest[2 * RING + 2 * NBUF:2 * RING + 3 * NBUF]
    agg = rest[2 * RING + 3 * NBUF]
    c = lax.axis_index("c")
    s = lax.axis_index("s")
    zrows = NPAD // NS
    pltpu.sync_copy(zeros_hbm, agg.at[pl.ds(s * zrows, zrows)])
    pltpu.sync_copy(dst_hbm.at[c, s], dst_v)

    def load_idx(j, i):
        pltpu.async_copy(idx_hbm.at[c, s, j], islots[i], isems[i])

    def wait_i(i):
        pltpu.make_async_copy(idx_hbm.at[c, s, 0], islots[i], isems[i]).wait()

    def gather(i, b):
        pltpu.async_copy(hw_hbm.at[islots[i]], bufs[b], gsems[b])

    def wait_g(b):
        pltpu.make_async_copy(hw_hbm.at[islots[0]], bufs[b], gsems[b]).wait()

    def scatter(j, b):
        pltpu.async_copy(bufs[b], agg.at[dst_v.at[j]], ssems[b], add=True)

    def wait_s(b):
        pltpu.make_async_copy(bufs[b], agg.at[dst_v.at[0]], ssems[b]).wait()

    plsc.subcore_barrier()

    for i in range(RING):
        load_idx(i, i)
    wait_i(0)
    gather(0, 0)

    def body(k, carry):
        for u in range(RING):
            j = k * RING + u

            wait_g(u % NBUF)

            @pl.when(j + RING < CH)
            def _():
                load_idx(j + RING, u)

            scatter(j, u % NBUF)

            @pl.when(j >= 1)
            def _():
                wait_s((u + 1) % NBUF)

            @pl.when(j + 1 < CH)
            def _():
                wait_i((u + 1) % RING)
                gather((u + 1) % RING, (u + 1) % NBUF)

        return carry

    lax.fori_loop(0, CH // RING, body, 0)
    wait_s((CH - 1) % NBUF)
    plsc.subcore_barrier()
    orows = NPAD // NS
    pltpu.sync_copy(agg.at[pl.ds(s * orows, orows)],
                    out_hbm.at[c, pl.ds(s * orows, orows)])


@functools.cache
def _get_sc_agg():
    return pl.kernel(
        _sc_body,
        out_type=jax.ShapeDtypeStruct((NC, NPAD, D), jnp.float32),
        mesh=plsc.VectorSubcoreMesh(core_axis_name="c", subcore_axis_name="s",
                                    num_cores=NC, num_subcores=NS),
        scratch_types=(
            [pltpu.VMEM((CH, LANE), jnp.int32)]
            + [pltpu.VMEM((LANE,), jnp.int32) for _ in range(RING)]
            + [pltpu.SemaphoreType.DMA for _ in range(RING)]
            + [pltpu.VMEM((LANE, D), jnp.float32) for _ in range(NBUF)]
            + [pltpu.SemaphoreType.DMA for _ in range(2 * NBUF)]
            + [pltpu.VMEM_SHARED((NPAD, D), jnp.float32)]
        ),
    )


def kernel(feat, edge_index, etypes,
           W0, bias0, loopW0, ln_g0, ln_b0,
           W1, bias1, loopW1, ln_g1, ln_b1,
           W2, bias2, loopW2, ln_g2, ln_b2):
    src = edge_index[0]
    dst = edge_index[1]
    flat = src * R + etypes
    nw = NC * NS
    rl = (E // nw) // CH
    npc = LANE - rl
    karange = jnp.arange(CH * npc, dtype=jnp.int32).reshape(CH, npc)
    idx_pad = jnp.broadcast_to((karange * 331) % (N * R), (nw, CH, npc))
    tile = (jnp.arange(nw, dtype=jnp.int32) % NS).reshape(nw, 1, 1)
    dst_pad = N + tile * ((NPAD - N) // NS) + karange % ((NPAD - N) // NS)
    flat_p = jnp.concatenate([flat.reshape(nw, CH, rl), idx_pad], axis=-1)
    dst_p = jnp.concatenate([dst.reshape(nw, CH, rl), dst_pad], axis=-1)
    idx3 = flat_p.reshape(NC, NS, CH, LANE)
    dst3 = dst_p.reshape(NC, NS, CH, LANE)
    zeros = jnp.zeros((NPAD // NS, D), jnp.float32)

    h = feat
    layers = (
        (W0, bias0, loopW0, ln_g0, ln_b0, True),
        (W1, bias1, loopW1, ln_g1, ln_b1, True),
        (W2, bias2, loopW2, ln_g2, ln_b2, False),
    )
    for W, bias, loopW, g, b, act in layers:
        hw, loop = _pre(h, W, loopW)
        parts = _get_sc_agg()(hw.reshape(N * R, D), idx3, dst3, zeros)
        post = _post_act if act else _post_noact
        h = post(parts, loop, g.reshape(1, D), b.reshape(1, D),
                 bias.reshape(1, D))
    return h
